# baseline (speedup 1.0000x reference)
import functools

import numpy as np
import jax
import jax.numpy as jnp
from jax import lax
from jax.experimental import pallas as pl
from jax.experimental.pallas import tpu as pltpu
from jax.experimental.pallas import tpu_sc as plsc

E = 8
K = 2
D = 1024
FF = 1408
T = 4096
R = T * K

RB = 512
NRB = T // RB

BLK = 1024
NBLKS = R // BLK + E
P = NBLKS * BLK

NW = 32
CS = 32
NCH = R // NW // CS
RW = NW * NCH
TW = T // NW
TG = 16


def _router_body(x_ref, rw_ref, e01_ref, rank_ref, wrep_ref, beo_ref,
                 starts_ref, carry):
    c = pl.program_id(0)

    @pl.when(c == 0)
    def _init():
        carry[...] = jnp.zeros_like(carry)

    x = x_ref[...]
    logits = lax.dot_general(x, rw_ref[...], (((1,), (1,)), ((), ())),
                             preferred_element_type=jnp.float32)
    m = jnp.max(logits, axis=-1, keepdims=True)
    ex = jnp.exp(logits - m)
    probs = ex / jnp.sum(ex, axis=-1, keepdims=True)

    iota8 = lax.broadcasted_iota(jnp.int32, (RB, E), 1)
    a1 = jnp.argmax(probs, axis=-1)
    p1 = jnp.max(probs, axis=-1)
    masked = jnp.where(iota8 == a1[:, None], -1.0, probs)
    a2 = jnp.argmax(masked, axis=-1)
    p2 = jnp.max(masked, axis=-1)

    lane32 = lax.broadcasted_iota(jnp.int32, (RB, 32), 1)
    wrep_ref[...] = jnp.where(lane32 < 16, p1[:, None], p2[:, None])
    e01_ref[...] = jnp.concatenate(
        [a1[:, None], a2[:, None]], axis=1).astype(jnp.int32)

    h0 = (iota8 == a1[:, None]).astype(jnp.float32)
    h1 = (iota8 == a2[:, None]).astype(jnp.float32)
    s = h0 + h1
    ri = lax.broadcasted_iota(jnp.int32, (RB, RB), 0)
    ci = lax.broadcasted_iota(jnp.int32, (RB, RB), 1)
    lstrict = (ri > ci).astype(jnp.float32)
    pex = lax.dot_general(lstrict, s, (((1,), (0,)), ((), ())),
                          preferred_element_type=jnp.float32)
    base = carry[...]
    msum = pex + base
    r0 = jnp.sum(jnp.where(iota8 == a1[:, None], msum, 0.0), axis=-1)
    r1 = jnp.sum(jnp.where(iota8 == a2[:, None], msum, 0.0), axis=-1)
    rank_ref[...] = jnp.concatenate(
        [r0[:, None], r1[:, None]], axis=1).astype(jnp.int32)
    newc = base + jnp.sum(s, axis=0, keepdims=True)
    carry[...] = newc

    @pl.when(c == NRB - 1)
    def _finalize():
        totals = newc
        nb = jnp.ceil(totals * (1.0 / BLK))
        er = lax.broadcasted_iota(jnp.int32, (E, E), 0)
        ec = lax.broadcasted_iota(jnp.int32, (E, E), 1)
        uinc = (er <= ec).astype(jnp.float32)
        cuminc = lax.dot_general(nb, uinc, (((1,), (0,)), ((), ())),
                                 preferred_element_type=jnp.float32)
        startrow = (cuminc - nb) * float(BLK)
        starts_ref[...] = jnp.concatenate(
            [startrow, startrow], axis=1).astype(jnp.int32)

        nb1 = NBLKS + 1
        bcol = lax.broadcasted_iota(jnp.int32, (nb1, E), 0)
        cmp = (cuminc.astype(jnp.int32) <= bcol).astype(jnp.int32)
        be = jnp.minimum(jnp.sum(cmp, axis=-1), E - 1)
        nused = jnp.sum(nb, dtype=jnp.float32).astype(jnp.int32)
        lanei = lax.broadcasted_iota(jnp.int32, (1, nb1), 1)
        beo_ref[...] = jnp.where(lanei == NBLKS, nused, be[None, :])


def _gemm_body(be_ref, x_ref, wg_ref, wu_ref, wd_ref, o_ref):
    b = pl.program_id(0)
    nused = be_ref[0, NBLKS]

    @pl.when(b < nused)
    def _compute():
        x = x_ref[...]
        gate = jnp.dot(x, wg_ref[0], preferred_element_type=jnp.float32)
        up = jnp.dot(x, wu_ref[0], preferred_element_type=jnp.float32)
        h = gate * lax.logistic(gate) * up
        o_ref[...] = jnp.dot(h, wd_ref[0], preferred_element_type=jnp.float32)


def _make_dispatch():
    mesh = plsc.VectorSubcoreMesh(core_axis_name="c", subcore_axis_name="s")

    @functools.partial(
        pl.kernel, mesh=mesh,
        out_type=[
            jax.ShapeDtypeStruct((P, D), jnp.float32),
            jax.ShapeDtypeStruct((RW, CS), jnp.int32),
        ],
        scratch_types=[
            pltpu.VMEM((NCH, CS), jnp.int32),
            pltpu.VMEM((NCH, CS), jnp.int32),
            pltpu.VMEM((NCH, CS), jnp.int32),
            pltpu.VMEM((NCH, CS), jnp.int32),
            pltpu.VMEM((16,), jnp.int32),
            pltpu.VMEM((CS, D), jnp.float32),
            pltpu.VMEM((CS, D), jnp.float32),
            pltpu.SemaphoreType.DMA,
            pltpu.SemaphoreType.DMA,
            pltpu.SemaphoreType.DMA,
        ],
    )
    def dispatch(hid_hbm, tok_hbm, e01_hbm, rank_hbm, starts_hbm,
                 perm_hbm, pos_hbm,
                 tok_v, e_v, r_v, pos_v, starts_v, rows_a, rows_b,
                 gsem_a, gsem_b, ssem):
        wid = lax.axis_index("s") * 2 + lax.axis_index("c")
        pltpu.sync_copy(tok_hbm.at[wid], tok_v)
        pltpu.sync_copy(e01_hbm.at[pl.ds(wid * NCH, NCH)], e_v)
        pltpu.sync_copy(rank_hbm.at[pl.ds(wid * NCH, NCH)], r_v)
        pltpu.sync_copy(starts_hbm.at[0], starts_v)
        s_reg = starts_v[0:16]
        dnums = lax.GatherDimensionNumbers(
            offset_dims=(), collapsed_slice_dims=(0,), start_index_map=(0,))
        for j in range(NCH):
            for u in range(CS // 16):
                sl = slice(u * 16, (u + 1) * 16)
                sv = lax.gather(s_reg, e_v[j, sl][:, None], dnums, (1,),
                                mode=lax.GatherScatterMode.PROMISE_IN_BOUNDS)
                pos_v[j, sl] = r_v[j, sl] + sv
        pltpu.sync_copy(pos_v, pos_hbm.at[pl.ds(wid * NCH, NCH)])

        bufs = (rows_a, rows_b)
        sems = (gsem_a, gsem_b)
        pending = [None, None]
        pending[0] = pltpu.async_copy(hid_hbm.at[tok_v.at[0]], bufs[0], sems[0])
        for j in range(NCH):
            cur = j % 2
            pending[cur].wait()
            if j + 1 < NCH:
                nxt = (j + 1) % 2
                pending[nxt] = pltpu.async_copy(
                    hid_hbm.at[tok_v.at[j + 1]], bufs[nxt], sems[nxt])
            pltpu.async_copy(bufs[cur], perm_hbm.at[pos_v.at[j]], ssem).wait()

    return dispatch


def _make_combine():
    mesh = plsc.VectorSubcoreMesh(core_axis_name="c", subcore_axis_name="s")

    @functools.partial(
        pl.kernel, mesh=mesh,
        out_type=jax.ShapeDtypeStruct((T, D), jnp.float32),
        scratch_types=[
            pltpu.VMEM((NCH, CS), jnp.int32),
            pltpu.VMEM((TW, 32), jnp.float32),
            pltpu.VMEM((CS, D), jnp.float32),
            pltpu.VMEM((CS, D), jnp.float32),
            pltpu.VMEM((TG, D), jnp.float32),
            pltpu.SemaphoreType.DMA,
            pltpu.SemaphoreType.DMA,
            pltpu.SemaphoreType.DMA,
        ],
    )
    def combine(operm_hbm, pos_hbm, wrep_hbm, out_hbm,
                pos_v, w_v, rows_a, rows_b, o_v, gsem_a, gsem_b, ssem):
        wid = lax.axis_index("s") * 2 + lax.axis_index("c")
        pltpu.sync_copy(pos_hbm.at[pl.ds(wid * NCH, NCH)], pos_v)
        pltpu.sync_copy(wrep_hbm.at[pl.ds(wid * TW, TW)], w_v)
        bufs = (rows_a, rows_b)
        sems = (gsem_a, gsem_b)
        pending = [None, None]
        pending[0] = pltpu.async_copy(
            operm_hbm.at[pos_v.at[0]], bufs[0], sems[0])
        for j in range(NCH):
            cur = j % 2
            pending[cur].wait()
            if j + 1 < NCH:
                nxt = (j + 1) % 2
                pending[nxt] = pltpu.async_copy(
                    operm_hbm.at[pos_v.at[j + 1]], bufs[nxt], sems[nxt])
            rows = bufs[cur]
            w0s = [w_v[j * TG + tt, 0:16] for tt in range(TG)]
            w1s = [w_v[j * TG + tt, 16:32] for tt in range(TG)]

            def cbody(ci, _, rows=rows, w0s=w0s, w1s=w1s):
                sl = pl.ds(ci * 16, 16)
                for tt in range(TG):
                    o_v[tt, sl] = (w0s[tt] * rows[2 * tt, sl]
                                   + w1s[tt] * rows[2 * tt + 1, sl])
                return 0

            lax.fori_loop(0, D // 16, cbody, 0)
            pltpu.sync_copy(o_v, out_hbm.at[pl.ds(wid * TW + j * TG, TG)])

    return combine


_TOK3 = np.repeat(np.arange(T, dtype=np.int32), K).reshape(NW, NCH, CS)


def kernel(hidden_states, router_weight, merged_gate_up_proj, merged_down_proj):
    e01, rank, wrep, beo, starts = pl.pallas_call(
        _router_body,
        grid=(NRB,),
        in_specs=[
            pl.BlockSpec((RB, D), lambda c: (c, 0)),
            pl.BlockSpec((E, D), lambda c: (0, 0)),
        ],
        out_specs=[
            pl.BlockSpec((RB, K), lambda c: (c, 0)),
            pl.BlockSpec((RB, K), lambda c: (c, 0)),
            pl.BlockSpec((RB, 32), lambda c: (c, 0)),
            pl.BlockSpec((1, NBLKS + 1), lambda c: (0, 0)),
            pl.BlockSpec((1, 16), lambda c: (0, 0)),
        ],
        out_shape=[
            jax.ShapeDtypeStruct((T, K), jnp.int32),
            jax.ShapeDtypeStruct((T, K), jnp.int32),
            jax.ShapeDtypeStruct((T, 32), jnp.float32),
            jax.ShapeDtypeStruct((1, NBLKS + 1), jnp.int32),
            jax.ShapeDtypeStruct((1, 16), jnp.int32),
        ],
        scratch_shapes=[pltpu.VMEM((1, E), jnp.float32)],
    )(hidden_states, router_weight)

    tok3 = jnp.asarray(_TOK3)
    perm, pos = _make_dispatch()(hidden_states, tok3,
                                 e01.reshape(RW, CS), rank.reshape(RW, CS),
                                 starts)

    grid_spec = pltpu.PrefetchScalarGridSpec(
        num_scalar_prefetch=1,
        grid=(NBLKS,),
        in_specs=[
            pl.BlockSpec((BLK, D), lambda b, be: (b, 0)),
            pl.BlockSpec((1, D, FF), lambda b, be: (be[0, b], 0, 0)),
            pl.BlockSpec((1, D, FF), lambda b, be: (be[0, b], 0, 1)),
            pl.BlockSpec((1, FF, D), lambda b, be: (be[0, b], 0, 0)),
        ],
        out_specs=pl.BlockSpec((BLK, D), lambda b, be: (b, 0)),
    )
    out_perm = pl.pallas_call(
        _gemm_body,
        grid_spec=grid_spec,
        out_shape=jax.ShapeDtypeStruct((P, D), jnp.float32),
    )(beo, perm, merged_gate_up_proj, merged_gate_up_proj,
      merged_down_proj)

    combined = _make_combine()(out_perm, pos, wrep)
    return combined


# speedup vs baseline: 1.0238x; 1.0238x over previous
import functools

import numpy as np
import jax
import jax.numpy as jnp
from jax import lax
from jax.experimental import pallas as pl
from jax.experimental.pallas import tpu as pltpu
from jax.experimental.pallas import tpu_sc as plsc

E = 8
K = 2
D = 1024
FF = 1408
T = 4096
R = T * K

RB = 1024
NRB = T // RB

BLK = 512
NBLKS = R // BLK + E
P = NBLKS * BLK

NW = 32
CS = 32
NCH = R // NW // CS
RW = NW * NCH
TW = T // NW
TG = 16


def _router_body(x_ref, rw_ref, e01_ref, rank_ref, wrep_ref, beo_ref,
                 starts_ref, carry):
    c = pl.program_id(0)

    @pl.when(c == 0)
    def _init():
        carry[...] = jnp.zeros_like(carry)

    x = x_ref[...]
    logits = lax.dot_general(x, rw_ref[...], (((1,), (1,)), ((), ())),
                             preferred_element_type=jnp.float32)
    m = jnp.max(logits, axis=-1, keepdims=True)
    ex = jnp.exp(logits - m)
    probs = ex / jnp.sum(ex, axis=-1, keepdims=True)

    iota8 = lax.broadcasted_iota(jnp.int32, (RB, E), 1)
    a1 = jnp.argmax(probs, axis=-1)
    p1 = jnp.max(probs, axis=-1)
    masked = jnp.where(iota8 == a1[:, None], -1.0, probs)
    a2 = jnp.argmax(masked, axis=-1)
    p2 = jnp.max(masked, axis=-1)

    lane32 = lax.broadcasted_iota(jnp.int32, (RB, 32), 1)
    wrep_ref[...] = jnp.where(lane32 < 16, p1[:, None], p2[:, None])
    e01_ref[...] = jnp.concatenate(
        [a1[:, None], a2[:, None]], axis=1).astype(jnp.int32)

    h0 = (iota8 == a1[:, None]).astype(jnp.float32)
    h1 = (iota8 == a2[:, None]).astype(jnp.float32)
    s = h0 + h1
    ri = lax.broadcasted_iota(jnp.int32, (RB, RB), 0)
    ci = lax.broadcasted_iota(jnp.int32, (RB, RB), 1)
    lstrict = (ri > ci).astype(jnp.float32)
    pex = lax.dot_general(lstrict, s, (((1,), (0,)), ((), ())),
                          preferred_element_type=jnp.float32)
    base = carry[...]
    msum = pex + base
    r0 = jnp.sum(jnp.where(iota8 == a1[:, None], msum, 0.0), axis=-1)
    r1 = jnp.sum(jnp.where(iota8 == a2[:, None], msum, 0.0), axis=-1)
    rank_ref[...] = jnp.concatenate(
        [r0[:, None], r1[:, None]], axis=1).astype(jnp.int32)
    newc = base + jnp.sum(s, axis=0, keepdims=True)
    carry[...] = newc

    @pl.when(c == NRB - 1)
    def _finalize():
        totals = newc
        nb = jnp.ceil(totals * (1.0 / BLK))
        er = lax.broadcasted_iota(jnp.int32, (E, E), 0)
        ec = lax.broadcasted_iota(jnp.int32, (E, E), 1)
        uinc = (er <= ec).astype(jnp.float32)
        cuminc = lax.dot_general(nb, uinc, (((1,), (0,)), ((), ())),
                                 preferred_element_type=jnp.float32)
        startrow = (cuminc - nb) * float(BLK)
        starts_ref[...] = jnp.concatenate(
            [startrow, startrow], axis=1).astype(jnp.int32)

        nb1 = NBLKS + 1
        bcol = lax.broadcasted_iota(jnp.int32, (nb1, E), 0)
        cmp = (cuminc.astype(jnp.int32) <= bcol).astype(jnp.int32)
        be = jnp.minimum(jnp.sum(cmp, axis=-1), E - 1)
        nused = jnp.sum(nb, dtype=jnp.float32).astype(jnp.int32)
        lanei = lax.broadcasted_iota(jnp.int32, (1, nb1), 1)
        beo_ref[...] = jnp.where(lanei == NBLKS, nused, be[None, :])


def _gemm_body(be_ref, x_ref, wg_ref, wu_ref, wd_ref, o_ref):
    b = pl.program_id(0)
    nused = be_ref[0, NBLKS]

    @pl.when(b < nused)
    def _compute():
        x = x_ref[...]
        gate = jnp.dot(x, wg_ref[0], preferred_element_type=jnp.float32)
        up = jnp.dot(x, wu_ref[0], preferred_element_type=jnp.float32)
        h = gate * lax.logistic(gate) * up
        o_ref[...] = jnp.dot(h, wd_ref[0], preferred_element_type=jnp.float32)


def _make_dispatch():
    mesh = plsc.VectorSubcoreMesh(core_axis_name="c", subcore_axis_name="s")

    @functools.partial(
        pl.kernel, mesh=mesh,
        out_type=[
            jax.ShapeDtypeStruct((P, D), jnp.float32),
            jax.ShapeDtypeStruct((RW, CS), jnp.int32),
        ],
        scratch_types=[
            pltpu.VMEM((NCH, CS), jnp.int32),
            pltpu.VMEM((NCH, CS), jnp.int32),
            pltpu.VMEM((NCH, CS), jnp.int32),
            pltpu.VMEM((NCH, CS), jnp.int32),
            pltpu.VMEM((16,), jnp.int32),
            pltpu.VMEM((CS, D), jnp.float32),
            pltpu.VMEM((CS, D), jnp.float32),
            pltpu.SemaphoreType.DMA,
            pltpu.SemaphoreType.DMA,
            pltpu.SemaphoreType.DMA,
            pltpu.SemaphoreType.DMA,
        ],
    )
    def dispatch(hid_hbm, tok_hbm, e01_hbm, rank_hbm, starts_hbm,
                 perm_hbm, pos_hbm,
                 tok_v, e_v, r_v, pos_v, starts_v, rows_a, rows_b,
                 gsem_a, gsem_b, ssem_a, ssem_b):
        wid = lax.axis_index("s") * 2 + lax.axis_index("c")
        pltpu.sync_copy(tok_hbm.at[wid], tok_v)
        pltpu.sync_copy(e01_hbm.at[pl.ds(wid * NCH, NCH)], e_v)
        pltpu.sync_copy(rank_hbm.at[pl.ds(wid * NCH, NCH)], r_v)
        pltpu.sync_copy(starts_hbm.at[0], starts_v)
        s_reg = starts_v[0:16]
        dnums = lax.GatherDimensionNumbers(
            offset_dims=(), collapsed_slice_dims=(0,), start_index_map=(0,))
        for j in range(NCH):
            for u in range(CS // 16):
                sl = slice(u * 16, (u + 1) * 16)
                sv = lax.gather(s_reg, e_v[j, sl][:, None], dnums, (1,),
                                mode=lax.GatherScatterMode.PROMISE_IN_BOUNDS)
                pos_v[j, sl] = r_v[j, sl] + sv
        pltpu.sync_copy(pos_v, pos_hbm.at[pl.ds(wid * NCH, NCH)])

        bufs = (rows_a, rows_b)
        sems = (gsem_a, gsem_b)
        ssems = (ssem_a, ssem_b)
        pending = [None, None]
        spending = [None, None]
        pending[0] = pltpu.async_copy(hid_hbm.at[tok_v.at[0]], bufs[0], sems[0])
        for j in range(NCH):
            cur = j % 2
            nxt = (j + 1) % 2
            pending[cur].wait()
            if j + 1 < NCH:
                if spending[nxt] is not None:
                    spending[nxt].wait()
                pending[nxt] = pltpu.async_copy(
                    hid_hbm.at[tok_v.at[j + 1]], bufs[nxt], sems[nxt])
            spending[cur] = pltpu.async_copy(
                bufs[cur], perm_hbm.at[pos_v.at[j]], ssems[cur])
        spending[0].wait()
        spending[1].wait()

    return dispatch


def _make_combine():
    mesh = plsc.VectorSubcoreMesh(core_axis_name="c", subcore_axis_name="s")

    @functools.partial(
        pl.kernel, mesh=mesh,
        out_type=jax.ShapeDtypeStruct((T, D), jnp.float32),
        scratch_types=[
            pltpu.VMEM((NCH, CS), jnp.int32),
            pltpu.VMEM((TW, 32), jnp.float32),
            pltpu.VMEM((CS, D), jnp.float32),
            pltpu.VMEM((CS, D), jnp.float32),
            pltpu.VMEM((TG, D), jnp.float32),
            pltpu.VMEM((TG, D), jnp.float32),
            pltpu.SemaphoreType.DMA,
            pltpu.SemaphoreType.DMA,
            pltpu.SemaphoreType.DMA,
            pltpu.SemaphoreType.DMA,
        ],
    )
    def combine(operm_hbm, pos_hbm, wrep_hbm, out_hbm,
                pos_v, w_v, rows_a, rows_b, o_a, o_b,
                gsem_a, gsem_b, ssem_a, ssem_b):
        wid = lax.axis_index("s") * 2 + lax.axis_index("c")
        pltpu.sync_copy(pos_hbm.at[pl.ds(wid * NCH, NCH)], pos_v)
        pltpu.sync_copy(wrep_hbm.at[pl.ds(wid * TW, TW)], w_v)
        bufs = (rows_a, rows_b)
        sems = (gsem_a, gsem_b)
        obufs = (o_a, o_b)
        osems = (ssem_a, ssem_b)
        pending = [None, None]
        spending = [None, None]
        pending[0] = pltpu.async_copy(
            operm_hbm.at[pos_v.at[0]], bufs[0], sems[0])
        for j in range(NCH):
            cur = j % 2
            pending[cur].wait()
            if j + 1 < NCH:
                nxt = (j + 1) % 2
                pending[nxt] = pltpu.async_copy(
                    operm_hbm.at[pos_v.at[j + 1]], bufs[nxt], sems[nxt])
            rows = bufs[cur]
            o_v = obufs[cur]
            if spending[cur] is not None:
                spending[cur].wait()
            w0s = [w_v[j * TG + tt, 0:16] for tt in range(TG)]
            w1s = [w_v[j * TG + tt, 16:32] for tt in range(TG)]

            def cbody(ci, _, rows=rows, w0s=w0s, w1s=w1s, o_v=o_v):
                sl = pl.ds(ci * 16, 16)
                for tt in range(TG):
                    o_v[tt, sl] = (w0s[tt] * rows[2 * tt, sl]
                                   + w1s[tt] * rows[2 * tt + 1, sl])
                return 0

            lax.fori_loop(0, D // 16, cbody, 0)
            spending[cur] = pltpu.async_copy(
                o_v, out_hbm.at[pl.ds(wid * TW + j * TG, TG)], osems[cur])
        spending[0].wait()
        spending[1].wait()

    return combine


_TOK3 = np.repeat(np.arange(T, dtype=np.int32), K).reshape(NW, NCH, CS)


def kernel(hidden_states, router_weight, merged_gate_up_proj, merged_down_proj):
    e01, rank, wrep, beo, starts = pl.pallas_call(
        _router_body,
        grid=(NRB,),
        in_specs=[
            pl.BlockSpec((RB, D), lambda c: (c, 0)),
            pl.BlockSpec((E, D), lambda c: (0, 0)),
        ],
        out_specs=[
            pl.BlockSpec((RB, K), lambda c: (c, 0)),
            pl.BlockSpec((RB, K), lambda c: (c, 0)),
            pl.BlockSpec((RB, 32), lambda c: (c, 0)),
            pl.BlockSpec((1, NBLKS + 1), lambda c: (0, 0)),
            pl.BlockSpec((1, 16), lambda c: (0, 0)),
        ],
        out_shape=[
            jax.ShapeDtypeStruct((T, K), jnp.int32),
            jax.ShapeDtypeStruct((T, K), jnp.int32),
            jax.ShapeDtypeStruct((T, 32), jnp.float32),
            jax.ShapeDtypeStruct((1, NBLKS + 1), jnp.int32),
            jax.ShapeDtypeStruct((1, 16), jnp.int32),
        ],
        scratch_shapes=[pltpu.VMEM((1, E), jnp.float32)],
    )(hidden_states, router_weight)

    tok3 = jnp.asarray(_TOK3)
    perm, pos = _make_dispatch()(hidden_states, tok3,
                                 e01.reshape(RW, CS), rank.reshape(RW, CS),
                                 starts)

    grid_spec = pltpu.PrefetchScalarGridSpec(
        num_scalar_prefetch=1,
        grid=(NBLKS,),
        in_specs=[
            pl.BlockSpec((BLK, D), lambda b, be: (b, 0)),
            pl.BlockSpec((1, D, FF), lambda b, be: (be[0, b], 0, 0)),
            pl.BlockSpec((1, D, FF), lambda b, be: (be[0, b], 0, 1)),
            pl.BlockSpec((1, FF, D), lambda b, be: (be[0, b], 0, 0)),
        ],
        out_specs=pl.BlockSpec((BLK, D), lambda b, be: (b, 0)),
    )
    out_perm = pl.pallas_call(
        _gemm_body,
        grid_spec=grid_spec,
        out_shape=jax.ShapeDtypeStruct((P, D), jnp.float32),
    )(beo, perm, merged_gate_up_proj, merged_gate_up_proj,
      merged_down_proj)

    combined = _make_combine()(out_perm, pos, wrep)
    return combined
